# baseline (speedup 1.0000x reference)
import functools

import jax
import jax.numpy as jnp
from jax import lax
from jax.experimental import pallas as pl
from jax.experimental.pallas import tpu as pltpu
from jax.experimental.pallas import tpu_sc as plsc

NC = 2
NS = 16
NW = NC * NS

R = 16384
S = 50
D = 32
DP = 128
R_PER_W = R // NW
CHUNK = 8
N_CHUNKS = R_PER_W // CHUNK

_mesh = plsc.VectorSubcoreMesh(
    core_axis_name="c", subcore_axis_name="s", num_cores=NC, num_subcores=NS
)


@functools.partial(
    pl.kernel,
    out_type=jax.ShapeDtypeStruct((R, S, D), jnp.float32),
    mesh=_mesh,
    scratch_types=[
        pltpu.VMEM((CHUNK, S), jnp.int32),
        pltpu.VMEM((CHUNK, S), jnp.int32),
        pltpu.VMEM((CHUNK, S, DP), jnp.float32),
        pltpu.VMEM((CHUNK, S, DP), jnp.float32),
        pltpu.SemaphoreType.DMA,
        pltpu.SemaphoreType.DMA,
        pltpu.SemaphoreType.DMA,
        pltpu.SemaphoreType.DMA,
        pltpu.SemaphoreType.DMA,
        pltpu.SemaphoreType.DMA,
    ],
    compiler_params=pltpu.CompilerParams(use_tc_tiling_on_sc=False),
)
def _gather_kernel(
    x_hbm, table_hbm, out_hbm,
    idx0, idx1, rows0, rows1, i0, i1, g0, g1, o0, o1
):
    wid = lax.axis_index("s") * NC + lax.axis_index("c")
    base = wid * R_PER_W

    def stage_idx(c, idx, sem):
        pltpu.async_copy(x_hbm.at[pl.ds(base + c * CHUNK, CHUNK)], idx, sem)

    def wait_idx(idx, sem):
        pltpu.make_async_copy(x_hbm.at[pl.ds(base, CHUNK)], idx, sem).wait()

    def fire(idx, rows, sem):
        for r in range(CHUNK):
            pltpu.async_copy(table_hbm.at[idx.at[r]], rows.at[r], sem)

    def wait_gathers(idx, rows, sem):
        for r in range(CHUNK):
            pltpu.make_async_copy(
                table_hbm.at[idx.at[r]], rows.at[r], sem
            ).wait()

    def writeback(c, rows, sem):
        pltpu.async_copy(
            rows.at[:, :, pl.ds(0, D)],
            out_hbm.at[pl.ds(base + c * CHUNK, CHUNK)],
            sem,
        )

    def wait_writeback(rows, sem):
        pltpu.make_async_copy(
            rows.at[:, :, pl.ds(0, D)], out_hbm.at[pl.ds(base, CHUNK)], sem
        ).wait()

    stage_idx(0, idx0, i0)
    stage_idx(1, idx1, i1)
    wait_idx(idx0, i0)
    fire(idx0, rows0, g0)

    @pl.loop(0, N_CHUNKS, step=2)
    def _pipeline(g):
        @pl.when(g > 0)
        def _():
            wait_writeback(rows1, o1)

        wait_idx(idx1, i1)
        fire(idx1, rows1, g1)
        wait_gathers(idx0, rows0, g0)
        writeback(g, rows0, o0)

        @pl.when(g + 2 < N_CHUNKS)
        def _():
            wait_writeback(rows0, o0)
            stage_idx(g + 2, idx0, i0)
            wait_idx(idx0, i0)
            fire(idx0, rows0, g0)

        wait_gathers(idx1, rows1, g1)
        writeback(g + 1, rows1, o1)

        @pl.when(g + 3 < N_CHUNKS)
        def _():
            stage_idx(g + 3, idx1, i1)

    wait_writeback(rows0, o0)
    wait_writeback(rows1, o1)


def kernel(x, table):
    tp = jnp.pad(table, ((0, 0), (0, DP - D)))
    return _gather_kernel(x, tp)


# speedup vs baseline: 1.0162x; 1.0162x over previous
import functools

import jax
import jax.numpy as jnp
from jax import lax
from jax.experimental import pallas as pl
from jax.experimental.pallas import tpu as pltpu
from jax.experimental.pallas import tpu_sc as plsc

NC = 2
NS = 16
NW = NC * NS

R = 16384
S = 50
D = 32
BLK = 128
R_PER_W = R // NW
N_BLKS = R_PER_W // BLK
L = 16

_mesh = plsc.VectorSubcoreMesh(
    core_axis_name="c", subcore_axis_name="s", num_cores=NC, num_subcores=NS
)


@functools.partial(
    pl.kernel,
    out_type=jax.ShapeDtypeStruct((S, D, R), jnp.float32),
    mesh=_mesh,
    scratch_types=[
        pltpu.VMEM((BLK, S), jnp.int32),
        pltpu.VMEM((S, BLK), jnp.int32),
        pltpu.VMEM((BLK, D), jnp.float32),
        pltpu.VMEM((BLK, D), jnp.float32),
        pltpu.VMEM((D, BLK), jnp.float32),
        pltpu.VMEM((D, BLK), jnp.float32),
        pltpu.SemaphoreType.DMA,
        pltpu.SemaphoreType.DMA,
        pltpu.SemaphoreType.DMA,
        pltpu.SemaphoreType.DMA,
        pltpu.SemaphoreType.DMA,
    ],
    compiler_params=pltpu.CompilerParams(
        use_tc_tiling_on_sc=False, needs_layout_passes=False
    ),
)
def _gather_kernel(
    x_hbm, table_hbm, out_hbm,
    xv, idxT, rows0, rows1, rT0, rT1, xs, g0, g1, o0, o1
):
    wid = lax.axis_index("s") * NC + lax.axis_index("c")
    base = wid * R_PER_W
    lanes = lax.iota(jnp.int32, L)

    def fire(j, rows, sem):
        pltpu.async_copy(table_hbm.at[idxT.at[j]], rows, sem)

    def wait_gather(rows, sem):
        pltpu.make_async_copy(table_hbm.at[idxT.at[0]], rows, sem).wait()

    def transpose_rows(rows, rT):
        @pl.loop(0, BLK, unroll=8)
        def _(t):
            t_ids = jnp.full((L,), t, jnp.int32)
            for k0 in (0, 16):
                v = plsc.load_gather(rows, [t_ids, k0 + lanes])
                plsc.store_scatter(rT, [k0 + lanes, t_ids], v)

    def writeback(j, i0, rT, sem):
        pltpu.async_copy(rT, out_hbm.at[j, :, pl.ds(i0, BLK)], sem)

    def wait_writeback(rT, sem):
        pltpu.make_async_copy(rT, out_hbm.at[0, :, pl.ds(0, BLK)], sem).wait()

    @pl.loop(0, N_BLKS)
    def _block(b):
        i0 = base + b * BLK
        pltpu.async_copy(x_hbm.at[pl.ds(i0, BLK)], xv, xs)
        pltpu.make_async_copy(x_hbm.at[pl.ds(0, BLK)], xv, xs).wait()

        @pl.loop(0, S)
        def _idx_t(j):
            j_ids = jnp.full((L,), j, jnp.int32)
            for l in range(BLK // L):
                v = plsc.load_gather(xv, [l * L + lanes, j_ids])
                plsc.store_scatter(idxT, [j_ids, l * L + lanes], v)

        fire(0, rows0, g0)

        @pl.loop(0, S, step=2)
        def _pipe(j):
            fire(j + 1, rows1, g1)
            wait_gather(rows0, g0)

            @pl.when(j > 0)
            def _():
                wait_writeback(rT0, o0)

            transpose_rows(rows0, rT0)
            writeback(j, i0, rT0, o0)

            @pl.when(j + 2 < S)
            def _():
                fire(j + 2, rows0, g0)

            wait_gather(rows1, g1)

            @pl.when(j > 0)
            def _():
                wait_writeback(rT1, o1)

            transpose_rows(rows1, rT1)
            writeback(j + 1, i0, rT1, o1)

        wait_writeback(rT0, o0)
        wait_writeback(rT1, o1)


def kernel(x, table):
    return _gather_kernel(x, table).transpose(2, 0, 1)


# speedup vs baseline: 1.3348x; 1.3135x over previous
import functools

import jax
import jax.numpy as jnp
from jax import lax
from jax.experimental import pallas as pl
from jax.experimental.pallas import tpu as pltpu
from jax.experimental.pallas import tpu_sc as plsc

NC = 2
NS = 16
NW = NC * NS

R = 16384
S = 50
D = 32
BLK = 128
R_PER_W = R // NW
N_BLKS = R_PER_W // BLK
L = 16
BLKP = 136

_mesh = plsc.VectorSubcoreMesh(
    core_axis_name="c", subcore_axis_name="s", num_cores=NC, num_subcores=NS
)


@functools.partial(
    pl.kernel,
    out_type=jax.ShapeDtypeStruct((S, D, R), jnp.float32),
    mesh=_mesh,
    scratch_types=[
        pltpu.VMEM((BLK, S), jnp.int32),
        pltpu.VMEM((S, BLKP), jnp.int32),
        pltpu.VMEM((BLK, D), jnp.float32),
        pltpu.VMEM((BLK, D), jnp.float32),
        pltpu.VMEM((D, BLKP), jnp.float32),
        pltpu.VMEM((D, BLKP), jnp.float32),
        pltpu.SemaphoreType.DMA,
        pltpu.SemaphoreType.DMA,
        pltpu.SemaphoreType.DMA,
        pltpu.SemaphoreType.DMA,
        pltpu.SemaphoreType.DMA,
    ],
    compiler_params=pltpu.CompilerParams(
        use_tc_tiling_on_sc=False, needs_layout_passes=False
    ),
)
def _gather_kernel(
    x_hbm, table_hbm, out_hbm,
    xv, idxT, rows0, rows1, rT0, rT1, xs, g0, g1, o0, o1
):
    wid = lax.axis_index("s") * NC + lax.axis_index("c")
    base = wid * R_PER_W
    lanes = lax.iota(jnp.int32, L)

    def fire(j, rows, sem):
        pltpu.async_copy(table_hbm.at[idxT.at[j, pl.ds(0, BLK)]], rows, sem)

    def wait_gather(rows, sem):
        pltpu.make_async_copy(
            table_hbm.at[idxT.at[0, pl.ds(0, BLK)]], rows, sem
        ).wait()

    def transpose_rows(rows, rT):
        @pl.loop(0, BLK, unroll=8)
        def _(t):
            t_ids = jnp.full((L,), t, jnp.int32)
            for k0 in (0, 16):
                v = plsc.load_gather(rows, [t_ids, k0 + lanes])
                plsc.store_scatter(rT, [k0 + lanes, t_ids], v)

    def writeback(j, i0, rT, sem):
        pltpu.async_copy(
            rT.at[:, pl.ds(0, BLK)], out_hbm.at[j, :, pl.ds(i0, BLK)], sem
        )

    def wait_writeback(rT, sem):
        pltpu.make_async_copy(
            rT.at[:, pl.ds(0, BLK)], out_hbm.at[0, :, pl.ds(0, BLK)], sem
        ).wait()

    @pl.loop(0, N_BLKS)
    def _block(b):
        i0 = base + b * BLK
        pltpu.async_copy(x_hbm.at[pl.ds(i0, BLK)], xv, xs)
        pltpu.make_async_copy(x_hbm.at[pl.ds(0, BLK)], xv, xs).wait()

        @pl.loop(0, S)
        def _idx_t(j):
            j_ids = jnp.full((L,), j, jnp.int32)
            for l in range(BLK // L):
                v = plsc.load_gather(xv, [l * L + lanes, j_ids])
                plsc.store_scatter(idxT, [j_ids, l * L + lanes], v)

        fire(0, rows0, g0)

        @pl.loop(0, S, step=2)
        def _pipe(j):
            fire(j + 1, rows1, g1)
            wait_gather(rows0, g0)

            @pl.when(j > 0)
            def _():
                wait_writeback(rT0, o0)

            transpose_rows(rows0, rT0)
            writeback(j, i0, rT0, o0)

            @pl.when(j + 2 < S)
            def _():
                fire(j + 2, rows0, g0)

            wait_gather(rows1, g1)

            @pl.when(j > 0)
            def _():
                wait_writeback(rT1, o1)

            transpose_rows(rows1, rT1)
            writeback(j + 1, i0, rT1, o1)

        wait_writeback(rT0, o0)
        wait_writeback(rT1, o1)


def kernel(x, table):
    return _gather_kernel(x, table).transpose(2, 0, 1)


# speedup vs baseline: 1.3735x; 1.0290x over previous
import functools

import jax
import jax.numpy as jnp
from jax import lax
from jax.experimental import pallas as pl
from jax.experimental.pallas import tpu as pltpu
from jax.experimental.pallas import tpu_sc as plsc

NC = 2
NS = 16
NW = NC * NS

R = 16384
S = 50
D = 32
BLK = 128
R_PER_W = R // NW
N_BLKS = R_PER_W // BLK
L = 16
BLKP = 136

_mesh = plsc.VectorSubcoreMesh(
    core_axis_name="c", subcore_axis_name="s", num_cores=NC, num_subcores=NS
)


@functools.partial(
    pl.kernel,
    out_type=jax.ShapeDtypeStruct((S, D, R), jnp.float32),
    mesh=_mesh,
    scratch_types=[
        pltpu.VMEM((BLK, S), jnp.int32),
        pltpu.VMEM((S, BLKP), jnp.int32),
        pltpu.VMEM((BLK, D), jnp.float32),
        pltpu.VMEM((BLK, D), jnp.float32),
        pltpu.VMEM((D, BLKP), jnp.float32),
        pltpu.VMEM((D, BLKP), jnp.float32),
        pltpu.SemaphoreType.DMA,
        pltpu.SemaphoreType.DMA,
        pltpu.SemaphoreType.DMA,
        pltpu.SemaphoreType.DMA,
        pltpu.SemaphoreType.DMA,
    ],
    compiler_params=pltpu.CompilerParams(
        use_tc_tiling_on_sc=False, needs_layout_passes=False
    ),
)
def _gather_kernel(
    x_hbm, table_hbm, out_hbm,
    xv, idxT, rows0, rows1, rT0, rT1, xs, g0, g1, o0, o1
):
    wid = lax.axis_index("s") * NC + lax.axis_index("c")
    base = wid * R_PER_W
    lanes = lax.iota(jnp.int32, L)

    def fire(j, rows, sem):
        pltpu.async_copy(table_hbm.at[idxT.at[j, pl.ds(0, BLK)]], rows, sem)

    def wait_gather(rows, sem):
        pltpu.make_async_copy(
            table_hbm.at[idxT.at[0, pl.ds(0, BLK)]], rows, sem
        ).wait()

    def transpose_rows(rows, rT):
        @pl.loop(0, BLK, unroll=16)
        def _(t):
            t_ids = jnp.full((L,), t, jnp.int32)
            for k0 in (0, 16):
                v = rows[t, pl.ds(k0, L)]
                plsc.store_scatter(rT, [k0 + lanes, t_ids], v)

    def writeback(j, i0, rT, sem):
        pltpu.async_copy(
            rT.at[:, pl.ds(0, BLK)], out_hbm.at[j, :, pl.ds(i0, BLK)], sem
        )

    def wait_writeback(rT, sem):
        pltpu.make_async_copy(
            rT.at[:, pl.ds(0, BLK)], out_hbm.at[0, :, pl.ds(0, BLK)], sem
        ).wait()

    @pl.loop(0, N_BLKS)
    def _block(b):
        i0 = base + b * BLK
        pltpu.async_copy(x_hbm.at[pl.ds(i0, BLK)], xv, xs)
        pltpu.make_async_copy(x_hbm.at[pl.ds(0, BLK)], xv, xs).wait()

        @pl.loop(0, S)
        def _idx_t(j):
            j_ids = jnp.full((L,), j, jnp.int32)
            for l in range(BLK // L):
                v = plsc.load_gather(xv, [l * L + lanes, j_ids])
                plsc.store_scatter(idxT, [j_ids, l * L + lanes], v)

        fire(0, rows0, g0)

        @pl.loop(0, S, step=2)
        def _pipe(j):
            fire(j + 1, rows1, g1)
            wait_gather(rows0, g0)

            @pl.when(j > 0)
            def _():
                wait_writeback(rT0, o0)

            transpose_rows(rows0, rT0)
            writeback(j, i0, rT0, o0)

            @pl.when(j + 2 < S)
            def _():
                fire(j + 2, rows0, g0)

            wait_gather(rows1, g1)

            @pl.when(j > 0)
            def _():
                wait_writeback(rT1, o1)

            transpose_rows(rows1, rT1)
            writeback(j + 1, i0, rT1, o1)

        wait_writeback(rT0, o0)
        wait_writeback(rT1, o1)


def kernel(x, table):
    return _gather_kernel(x, table).transpose(2, 0, 1)
